# baseline (speedup 1.0000x reference)
import functools

import jax
import jax.numpy as jnp
from jax import lax
from jax.experimental import pallas as pl
from jax.experimental.pallas import tpu as pltpu
from jax.experimental.pallas import tpu_sc as plsc

N = 10000
K = 32
F = 128
L = 16
NF = F // L

NC = 2
NS = 16
NW = NC * NS

CB = 4
GI = CB * K
NCHUNK = N // CB
CW = 80
CLAST = NCHUNK - (NW - 1) * CW
STRIPE = 624
TAIL = N - NS * STRIPE


def _sc_body(x_hbm, idx_hbm, out_hbm, x_sp, idx_all, neigh, out_v,
             gsem0, gsem1, osem0, osem1):
    sid = lax.axis_index("s")
    wid = sid * NC + lax.axis_index("c")
    cb0 = wid * CW
    ncw = jnp.where(wid == NW - 1, CLAST, CW)

    pltpu.sync_copy(x_hbm.at[pl.ds(sid * STRIPE, STRIPE)],
                    x_sp.at[pl.ds(sid * STRIPE, STRIPE)])
    pl.when(sid == NS - 1)(lambda: pltpu.sync_copy(
        x_hbm.at[pl.ds(NS * STRIPE, TAIL)],
        x_sp.at[pl.ds(NS * STRIPE, TAIL)]))

    pl.when(wid < NW - 1)(lambda: pltpu.sync_copy(
        idx_hbm.at[pl.ds(cb0 * GI, CW * GI)], idx_all.at[pl.ds(0, CW * GI)]))
    pl.when(wid == NW - 1)(lambda: pltpu.sync_copy(
        idx_hbm.at[pl.ds(cb0 * GI, CLAST * GI)],
        idx_all.at[pl.ds(0, CLAST * GI)]))
    plsc.subcore_barrier()

    gsems = (gsem0, gsem1)
    osems = (osem0, osem1)

    def start_gather(slot, c):
        pltpu.make_async_copy(
            x_sp.at[idx_all.at[pl.ds(c * GI, GI)]], neigh.at[slot],
            gsems[slot]).start()

    def wait_gather(slot, c):
        pltpu.make_async_copy(
            x_sp.at[idx_all.at[pl.ds(c * GI, GI)]], neigh.at[slot],
            gsems[slot]).wait()

    def start_write(slot, g):
        pltpu.make_async_copy(
            out_v.at[slot], out_hbm.at[pl.ds(g * CB, CB)],
            osems[slot]).start()

    def wait_write(slot, g):
        pltpu.make_async_copy(
            out_v.at[slot], out_hbm.at[pl.ds(g * CB, CB)],
            osems[slot]).wait()

    inv_k = jnp.float32(1.0 / K)

    UK = 8

    def compute(slot, c):
        for d in range(CB):
            def kbody(kk, c2, d=d):
                sums, maxs = c2
                for u in range(UK):
                    r = d * K + kk * UK + u
                    for f in range(NF):
                        v = neigh[slot, r, pl.ds(f * L, L)]
                        sums = sums[:f] + (sums[f] + v,) + sums[f + 1:]
                        maxs = maxs[:f] + (jnp.maximum(maxs[f], v),) + maxs[f + 1:]
                return sums, maxs

            z = tuple(jnp.zeros((L,), jnp.float32) for _ in range(NF))
            ninf = tuple(jnp.full((L,), -jnp.inf, jnp.float32) for _ in range(NF))
            sums, maxs = lax.fori_loop(0, K // UK, kbody, (z, ninf))
            for f in range(NF):
                out_v[slot, d, pl.ds(f * L, L)] = sums[f] * inv_k
                out_v[slot, d, pl.ds(F + f * L, L)] = maxs[f]

    start_gather(0, 0)
    start_gather(1, 1)

    def body(t, carry):
        for slot in range(2):
            c = 2 * t + slot
            wait_gather(slot, c)
            pl.when(t >= 1)(lambda slot=slot, c=c: wait_write(slot, cb0 + c - 2))
            compute(slot, c)
            start_write(slot, cb0 + c)
            pl.when(c + 2 < ncw)(
                lambda slot=slot, c=c: start_gather(slot, c + 2))
        return carry

    lax.fori_loop(0, ncw // 2, body, 0)

    wait_write(0, cb0 + ncw - 2)
    wait_write(1, cb0 + ncw - 1)


@jax.jit
def _run(x, idx2d):
    mesh = plsc.VectorSubcoreMesh(
        core_axis_name="c", subcore_axis_name="s",
        num_cores=NC, num_subcores=NS,
    )
    kfn = pl.kernel(
        _sc_body,
        out_type=jax.ShapeDtypeStruct((N, 2 * F), jnp.float32),
        mesh=mesh,
        scratch_types=[
            pltpu.VMEM_SHARED((N, F), jnp.float32),
            pltpu.VMEM((CW * GI,), jnp.int32),
            pltpu.VMEM((2, GI, F), jnp.float32),
            pltpu.VMEM((2, CB, 2 * F), jnp.float32),
            pltpu.SemaphoreType.DMA,
            pltpu.SemaphoreType.DMA,
            pltpu.SemaphoreType.DMA,
            pltpu.SemaphoreType.DMA,
        ],
    )
    return kfn(x, idx2d)


def kernel(x, idxs):
    return _run(x, idxs.reshape(-1))


# speedup vs baseline: 1.0196x; 1.0196x over previous
import functools

import jax
import jax.numpy as jnp
from jax import lax
from jax.experimental import pallas as pl
from jax.experimental.pallas import tpu as pltpu
from jax.experimental.pallas import tpu_sc as plsc

N = 10000
K = 32
F = 128
L = 16
NF = F // L

NC = 2
NS = 16
NW = NC * NS

CB = 4
GI = CB * K
NCHUNK = N // CB
CW = 80
CLAST = NCHUNK - (NW - 1) * CW
STRIPE = 624
TAIL = N - NS * STRIPE


def _sc_body(x_hbm, idx_hbm, out_hbm, x_sp, idx_all, neigh, out_v,
             gsem0, gsem1, osem0, osem1):
    sid = lax.axis_index("s")
    wid = sid * NC + lax.axis_index("c")
    cb0 = wid * CW
    ncw = jnp.where(wid == NW - 1, CLAST, CW)

    pltpu.make_async_copy(x_hbm.at[pl.ds(sid * STRIPE, STRIPE)],
                          x_sp.at[pl.ds(sid * STRIPE, STRIPE)], gsem0).start()
    pl.when(sid == NS - 1)(lambda: pltpu.make_async_copy(
        x_hbm.at[pl.ds(NS * STRIPE, TAIL)],
        x_sp.at[pl.ds(NS * STRIPE, TAIL)], gsem1).start())
    pl.when(wid < NW - 1)(lambda: pltpu.make_async_copy(
        idx_hbm.at[pl.ds(cb0 * GI, CW * GI)], idx_all.at[pl.ds(0, CW * GI)],
        osem0).start())
    pl.when(wid == NW - 1)(lambda: pltpu.make_async_copy(
        idx_hbm.at[pl.ds(cb0 * GI, CLAST * GI)],
        idx_all.at[pl.ds(0, CLAST * GI)], osem0).start())
    pltpu.make_async_copy(x_hbm.at[pl.ds(sid * STRIPE, STRIPE)],
                          x_sp.at[pl.ds(sid * STRIPE, STRIPE)], gsem0).wait()
    pl.when(sid == NS - 1)(lambda: pltpu.make_async_copy(
        x_hbm.at[pl.ds(NS * STRIPE, TAIL)],
        x_sp.at[pl.ds(NS * STRIPE, TAIL)], gsem1).wait())
    pl.when(wid < NW - 1)(lambda: pltpu.make_async_copy(
        idx_hbm.at[pl.ds(cb0 * GI, CW * GI)], idx_all.at[pl.ds(0, CW * GI)],
        osem0).wait())
    pl.when(wid == NW - 1)(lambda: pltpu.make_async_copy(
        idx_hbm.at[pl.ds(cb0 * GI, CLAST * GI)],
        idx_all.at[pl.ds(0, CLAST * GI)], osem0).wait())
    plsc.subcore_barrier()

    gsems = (gsem0, gsem1)
    osems = (osem0, osem1)

    def start_gather(slot, c):
        pltpu.make_async_copy(
            x_sp.at[idx_all.at[pl.ds(c * GI, GI)]], neigh.at[slot],
            gsems[slot]).start()

    def wait_gather(slot, c):
        pltpu.make_async_copy(
            x_sp.at[idx_all.at[pl.ds(c * GI, GI)]], neigh.at[slot],
            gsems[slot]).wait()

    def start_write(slot, g):
        pltpu.make_async_copy(
            out_v.at[slot], out_hbm.at[pl.ds(g * CB, CB)],
            osems[slot]).start()

    def wait_write(slot, g):
        pltpu.make_async_copy(
            out_v.at[slot], out_hbm.at[pl.ds(g * CB, CB)],
            osems[slot]).wait()

    inv_k = jnp.float32(1.0 / K)

    UK = 4

    def compute(slot, c):
        for d in range(CB):
            def kbody(kk, c2, d=d):
                sums, maxs = c2
                for u in range(UK):
                    r = d * K + kk * UK + u
                    for f in range(NF):
                        v = neigh[slot, r, pl.ds(f * L, L)]
                        sums = sums[:f] + (sums[f] + v,) + sums[f + 1:]
                        maxs = maxs[:f] + (jnp.maximum(maxs[f], v),) + maxs[f + 1:]
                return sums, maxs

            z = tuple(jnp.zeros((L,), jnp.float32) for _ in range(NF))
            ninf = tuple(jnp.full((L,), -jnp.inf, jnp.float32) for _ in range(NF))
            sums, maxs = lax.fori_loop(0, K // UK, kbody, (z, ninf))
            for f in range(NF):
                out_v[slot, d, pl.ds(f * L, L)] = sums[f] * inv_k
                out_v[slot, d, pl.ds(F + f * L, L)] = maxs[f]

    start_gather(0, 0)
    start_gather(1, 1)

    def body(t, carry):
        for slot in range(2):
            c = 2 * t + slot
            wait_gather(slot, c)
            pl.when(t >= 1)(lambda slot=slot, c=c: wait_write(slot, cb0 + c - 2))
            compute(slot, c)
            start_write(slot, cb0 + c)
            pl.when(c + 2 < ncw)(
                lambda slot=slot, c=c: start_gather(slot, c + 2))
        return carry

    lax.fori_loop(0, ncw // 2, body, 0)

    wait_write(0, cb0 + ncw - 2)
    wait_write(1, cb0 + ncw - 1)


@jax.jit
def _run(x, idx2d):
    mesh = plsc.VectorSubcoreMesh(
        core_axis_name="c", subcore_axis_name="s",
        num_cores=NC, num_subcores=NS,
    )
    kfn = pl.kernel(
        _sc_body,
        out_type=jax.ShapeDtypeStruct((N, 2 * F), jnp.float32),
        mesh=mesh,
        scratch_types=[
            pltpu.VMEM_SHARED((N, F), jnp.float32),
            pltpu.VMEM((CW * GI,), jnp.int32),
            pltpu.VMEM((2, GI, F), jnp.float32),
            pltpu.VMEM((2, CB, 2 * F), jnp.float32),
            pltpu.SemaphoreType.DMA,
            pltpu.SemaphoreType.DMA,
            pltpu.SemaphoreType.DMA,
            pltpu.SemaphoreType.DMA,
        ],
    )
    return kfn(x, idx2d)


def kernel(x, idxs):
    return _run(x, idxs.reshape(-1))
